# baseline (speedup 1.0000x reference)
import functools

import jax
import jax.numpy as jnp
from jax import lax
from jax.experimental import pallas as pl
from jax.experimental.pallas import tpu as pltpu
from jax.experimental.pallas import tpu_sc as plsc

H = 128
NUM_CLASSES = 10
NUM_LAYERS = 3
N_GRAPHS = 64
N_COMP, N_PIN, N_NET, N_SUB = 20000, 50000, 20000, 2000

RELS = (
    ("cp", "component", "pin", 160000),
    ("pc", "pin", "component", 160000),
    ("sp", "subcircuit", "pin", 20000),
    ("ps", "pin", "subcircuit", 20000),
    ("pn", "pin", "net", 120000),
    ("np", "net", "pin", 120000),
)
N_NODES = {"component": N_COMP, "pin": N_PIN, "net": N_NET, "subcircuit": N_SUB}
CHUNKS = {
    "pin": (4, 12544),
    "component": (2, 10112),
    "net": (2, 10112),
    "subcircuit": (2, 1024),
}
NDP = {t: nc * c for t, (nc, c) in CHUNKS.items()}
DST_RELS = {
    "pin": (0, 2, 5),
    "component": (1,),
    "subcircuit": (3,),
    "net": (4,),
}

ACC_ROWS = 12560
BATCH = 128
EBLK = 2048
NTILES = 16


def _epad(e):
    return ((e + 255) // 256) * 256


def _chunk_loop(mode, eids, cid, sid, rel_bufs, zero_fn, fire_fn, write_fn,
                src_blk, dst_blk, src_stage, dstl_stage):
    for r, (_, st, dt, ne) in enumerate(RELS):
        s_hbm, d_hbm = eids[r]
        nc, C = CHUNKS[dt]
        ncs = nc // 2
        rpt = C // NTILES
        share = _epad(ne) // NTILES

        def refill():
            ts = jnp.full((16,), C, jnp.int32)
            zs = jnp.zeros((16,), jnp.int32)
            for j in range(BATCH // 16):
                dstl_stage[pl.ds(j * 16, 16)] = ts
                if mode == "sum":
                    src_stage[pl.ds(j * 16, 16)] = zs

        def fire():
            fire_fn(r, rel_bufs[r])
            refill()

        for cl in range(ncs):
            chunk = cid * ncs + cl
            c0 = chunk * C
            c1 = c0 + C
            base_row = sid * rpt
            zero_fn(base_row, rpt)
            plsc.subcore_barrier()

            refill()
            cursor = jnp.int32(0)
            for blk in range(0, share, EBLK):
                sz = min(EBLK, share - blk)
                ebase = sid * share + blk
                if mode == "sum":
                    pltpu.sync_copy(s_hbm.at[pl.ds(ebase, sz)],
                                    src_blk.at[pl.ds(0, sz)])
                pltpu.sync_copy(d_hbm.at[pl.ds(ebase, sz)],
                                dst_blk.at[pl.ds(0, sz)])

                def step(i, cursor):
                    need = cursor > BATCH - 16
                    @pl.when(need)
                    def _():
                        fire()
                    cursor = jnp.where(need, 0, cursor)
                    d = dst_blk[pl.ds(i * 16, 16)]
                    m = (d >= c0) & (d < c1)
                    if mode == "sum":
                        s = src_blk[pl.ds(i * 16, 16)]
                        plsc.store_compressed(
                            src_stage.at[pl.ds(cursor, 16)], s, mask=m)
                    plsc.store_compressed(dstl_stage.at[pl.ds(cursor, 16)],
                                          d - c0, mask=m)
                    return cursor + jnp.sum(m.astype(jnp.int32))

                cursor = lax.fori_loop(0, sz // 16, step, cursor)

            fire()
            plsc.subcore_barrier()
            write_fn(r, rel_bufs[r], c0 + base_row, base_row, rpt)
            plsc.subcore_barrier()


def _sum_body(*refs):
    it = iter(refs)
    x_of = {t: next(it) for t in ("component", "pin", "net", "subcircuit")}
    eids = [(next(it), next(it)) for _ in RELS]
    z128 = next(it)
    accs = [next(it) for _ in RELS]
    acc_sh = next(it)
    src_blk = next(it)
    dst_blk = next(it)
    src_stage = next(it)
    dstl_stage = next(it)
    rows_buf = next(it)
    sem = next(it)
    cid = lax.axis_index("c")
    sid = lax.axis_index("s")

    def zero_fn(base_row, rpt):
        off = 0
        while off < rpt:
            p = min(128, rpt - off)
            pltpu.sync_copy(z128.at[pl.ds(0, p)],
                            acc_sh.at[pl.ds(base_row + off, p)])
            off += p

    def fire_fn(r, xsrc):
        pltpu.async_copy(xsrc.at[src_stage], rows_buf, sem).wait()
        pltpu.sync_copy(rows_buf, acc_sh.at[dstl_stage], add=True)

    def write_fn(r, xsrc, orow, srow, rpt):
        pltpu.sync_copy(acc_sh.at[pl.ds(srow, rpt)],
                        accs[r].at[pl.ds(orow, rpt)])

    xsrcs = [x_of[st] for (_, st, _, _) in RELS]
    _chunk_loop("sum", eids, cid, sid, xsrcs, zero_fn, fire_fn, write_fn,
                src_blk, dst_blk, src_stage, dstl_stage)


def _count_body(*refs):
    it = iter(refs)
    eids = [(next(it), next(it)) for _ in RELS]
    z128 = next(it)
    o128 = next(it)
    cnts = [next(it) for _ in RELS]
    cnt_sh = next(it)
    dst_blk = next(it)
    dstl_stage = next(it)
    ones_buf = next(it)
    sem = next(it)
    cid = lax.axis_index("c")
    sid = lax.axis_index("s")
    pltpu.sync_copy(o128, ones_buf)

    def zero_fn(base_row, rpt):
        off = 0
        while off < rpt:
            p = min(128, rpt - off)
            pltpu.sync_copy(z128.at[pl.ds(0, p)],
                            cnt_sh.at[pl.ds(base_row + off, p)])
            off += p

    def fire_fn(r, _):
        pltpu.sync_copy(ones_buf, cnt_sh.at[dstl_stage], add=True)

    def write_fn(r, _, orow, srow, rpt):
        pltpu.sync_copy(cnt_sh.at[pl.ds(srow, rpt)],
                        cnts[r].at[pl.ds(orow, rpt)])

    _chunk_loop("count", eids, cid, sid, [None] * len(RELS), zero_fn,
                fire_fn, write_fn, None, dst_blk, None, dstl_stage)


_sum_kernel = pl.kernel(
    _sum_body,
    out_type=tuple(jax.ShapeDtypeStruct((NDP[dt], H), jnp.float32)
                   for (_, _, dt, _) in RELS),
    mesh=plsc.VectorSubcoreMesh(core_axis_name="c", subcore_axis_name="s"),
    compiler_params=pltpu.CompilerParams(needs_layout_passes=False),
    scratch_types=[
        pltpu.VMEM_SHARED((ACC_ROWS, H), jnp.float32),
        pltpu.VMEM((EBLK,), jnp.int32),
        pltpu.VMEM((EBLK,), jnp.int32),
        pltpu.VMEM((BATCH,), jnp.int32),
        pltpu.VMEM((BATCH,), jnp.int32),
        pltpu.VMEM((BATCH, H), jnp.float32),
        pltpu.SemaphoreType.DMA,
    ],
)

_count_kernel = pl.kernel(
    _count_body,
    out_type=tuple(jax.ShapeDtypeStruct((NDP[dt], H), jnp.float32)
                   for (_, _, dt, _) in RELS),
    mesh=plsc.VectorSubcoreMesh(core_axis_name="c", subcore_axis_name="s"),
    compiler_params=pltpu.CompilerParams(needs_layout_passes=False),
    scratch_types=[
        pltpu.VMEM_SHARED((ACC_ROWS, H), jnp.float32),
        pltpu.VMEM((EBLK,), jnp.int32),
        pltpu.VMEM((BATCH,), jnp.int32),
        pltpu.VMEM((BATCH, H), jnp.float32),
        pltpu.SemaphoreType.DMA,
    ],
)


NC_PAD = 20224
PG = 72


def _pool_body(feat, batch_hbm, sums, maxs, cnts,
               feat_blk, bid_blk, acc_sum, acc_max, acc_cnt, sem):
    cid = lax.axis_index("c")
    sid = lax.axis_index("s")
    wid = sid * 2 + cid
    rows = NC_PAD // 32

    def init_row(i, _):
        g = i // 8
        o = (i % 8) * 16
        acc_sum[g, pl.ds(o, 16)] = jnp.zeros((16,), jnp.float32)
        acc_max[g, pl.ds(o, 16)] = jnp.full((16,), -1e30, jnp.float32)
        return 0

    lax.fori_loop(0, PG * 8, init_row, 0)

    def init_cnt(i, _):
        acc_cnt[i, pl.ds(0, 16)] = jnp.zeros((16,), jnp.float32)
        return 0

    lax.fori_loop(0, PG, init_cnt, 0)

    base = wid * rows
    pltpu.sync_copy(batch_hbm.at[pl.ds(base, rows)], bid_blk.at[pl.ds(0, rows)])
    done = 0
    while done < rows:
        nb = min(64, rows - done)
        pltpu.sync_copy(feat.at[pl.ds(base + done, nb)],
                        feat_blk.at[pl.ds(0, nb)])

        def row(i, _):
            g = bid_blk[pl.ds(done + i, 16)][0]
            acc_cnt[g, pl.ds(0, 16)] = acc_cnt[g, pl.ds(0, 16)] + 1.0
            for j in range(8):
                o = j * 16
                v = feat_blk[i, pl.ds(o, 16)]
                acc_sum[g, pl.ds(o, 16)] = acc_sum[g, pl.ds(o, 16)] + v
                acc_max[g, pl.ds(o, 16)] = jnp.maximum(
                    acc_max[g, pl.ds(o, 16)], v)
            return 0

        lax.fori_loop(0, nb, row, 0)
        done += nb

    pltpu.sync_copy(acc_sum.at[pl.ds(0, N_GRAPHS)], sums.at[wid])
    pltpu.sync_copy(acc_max.at[pl.ds(0, N_GRAPHS)], maxs.at[wid])
    pltpu.sync_copy(acc_cnt.at[pl.ds(0, N_GRAPHS)], cnts.at[wid])


_pool_kernel = pl.kernel(
    _pool_body,
    out_type=(
        jax.ShapeDtypeStruct((32, N_GRAPHS, H), jnp.float32),
        jax.ShapeDtypeStruct((32, N_GRAPHS, H), jnp.float32),
        jax.ShapeDtypeStruct((32, N_GRAPHS, 16), jnp.float32),
    ),
    mesh=plsc.VectorSubcoreMesh(core_axis_name="c", subcore_axis_name="s"),
    compiler_params=pltpu.CompilerParams(needs_layout_passes=False),
    scratch_types=[
        pltpu.VMEM((64, H), jnp.float32),
        pltpu.VMEM((648,), jnp.int32),
        pltpu.VMEM((PG, H), jnp.float32),
        pltpu.VMEM((PG, H), jnp.float32),
        pltpu.VMEM((PG, 16), jnp.float32),
        pltpu.SemaphoreType.DMA,
    ],
)


RB = 512


def _emb_tc_body(nt_ref, ct_ref, pt_ref, tbl_ref, out_ref):
    b = nt_ref.shape[0]
    io = lax.broadcasted_iota(jnp.int32, (b, 32), 1)
    nt = nt_ref[...]
    ct = jnp.maximum(ct_ref[...], 0) + 4
    pt = jnp.maximum(pt_ref[...], 0) + 13
    oh = ((io == nt).astype(jnp.float32) + (io == ct).astype(jnp.float32)
          + (io == pt).astype(jnp.float32))
    out_ref[...] = jnp.dot(oh, tbl_ref[...],
                           preferred_element_type=jnp.float32)


def _embed_tc(nt, ct, pt, tbl, n):
    grid = pl.cdiv(n, RB)
    return pl.pallas_call(
        _emb_tc_body,
        grid=(grid,),
        in_specs=[
            pl.BlockSpec((RB, 1), lambda i: (i, 0)),
            pl.BlockSpec((RB, 1), lambda i: (i, 0)),
            pl.BlockSpec((RB, 1), lambda i: (i, 0)),
            pl.BlockSpec((32, H), lambda i: (0, 0)),
        ],
        out_specs=pl.BlockSpec((RB, H), lambda i: (i, 0)),
        out_shape=jax.ShapeDtypeStruct((n, H), jnp.float32),
    )(nt, ct, pt, tbl)


def _self_tc_body(x_ref, w_ref, b_ref, out_ref):
    out_ref[...] = jnp.dot(x_ref[...], w_ref[...],
                           preferred_element_type=jnp.float32) + b_ref[...]


def _self_tc(x, w, b, n):
    grid = pl.cdiv(n, RB)
    return pl.pallas_call(
        _self_tc_body,
        grid=(grid,),
        in_specs=[
            pl.BlockSpec((RB, H), lambda i: (i, 0)),
            pl.BlockSpec((H, H), lambda i: (0, 0)),
            pl.BlockSpec((1, H), lambda i: (0, 0)),
        ],
        out_specs=pl.BlockSpec((RB, H), lambda i: (i, 0)),
        out_shape=jax.ShapeDtypeStruct((n, H), jnp.float32),
    )(x, w, b)


def _mix_tc_body(k, *refs):
    s_ref = refs[0]
    out_ref = refs[-1]
    h = s_ref[...]
    for i in range(k):
        a_ref = refs[1 + 2 * i]
        c_ref = refs[2 + 2 * i]
        w_ref = refs[1 + 2 * k + i]
        inv = 1.0 / jnp.maximum(c_ref[...][:, 0:1], 1.0)
        h = h + jnp.dot(a_ref[...] * inv, w_ref[...],
                        preferred_element_type=jnp.float32)
    out_ref[...] = jnp.maximum(h, 0.0)


def _mix_tc(s, accs, cnts, wls, n):
    k = len(accs)
    grid = pl.cdiv(n, RB)
    in_specs = [pl.BlockSpec((RB, H), lambda i: (i, 0))]
    for _ in range(k):
        in_specs.append(pl.BlockSpec((RB, H), lambda i: (i, 0)))
        in_specs.append(pl.BlockSpec((RB, H), lambda i: (i, 0)))
    for _ in range(k):
        in_specs.append(pl.BlockSpec((H, H), lambda i: (0, 0)))
    args = [s]
    for a, c in zip(accs, cnts):
        args += [a, c]
    args += list(wls)
    return pl.pallas_call(
        functools.partial(_mix_tc_body, k),
        grid=(grid,),
        in_specs=in_specs,
        out_specs=pl.BlockSpec((RB, H), lambda i: (i, 0)),
        out_shape=jax.ShapeDtypeStruct((n, H), jnp.float32),
    )(*args)


def _mlp_body(sums_ref, maxs_ref, cnts_ref, W1_ref, b1_ref, Wr1_ref, br1_ref,
              Wr2_ref, br2_ref, W2_ref, b2_ref, W3_ref, b3_ref, out_ref):
    sp = sums_ref[0]
    mp = maxs_ref[0]
    cp = cnts_ref[0]
    for i in range(1, 32):
        sp = sp + sums_ref[i]
        mp = jnp.maximum(mp, maxs_ref[i])
        cp = cp + cnts_ref[i]
    inv = 1.0 / jnp.maximum(cp[:, 0:1], 1.0)
    g = jnp.concatenate([sp * inv, mp], axis=1)
    h = jnp.dot(g, W1_ref[...], preferred_element_type=jnp.float32) + b1_ref[...]
    res = h
    y = jnp.maximum(jnp.dot(h, Wr1_ref[...],
                            preferred_element_type=jnp.float32) + br1_ref[...],
                    0.0)
    y = jnp.dot(y, Wr2_ref[...],
                preferred_element_type=jnp.float32) + br2_ref[...] + res
    hh = jnp.maximum(y, 0.0)
    hh = jnp.maximum(jnp.dot(hh, W2_ref[...],
                             preferred_element_type=jnp.float32) + b2_ref[...],
                     0.0)
    out_ref[...] = jnp.dot(hh, W3_ref[...],
                           preferred_element_type=jnp.float32) + b3_ref[...]


def _mlp_tc(sums, maxs, cnts, W1, b1, Wr1, br1, Wr2, br2, W2, b2, W3, b3):
    return pl.pallas_call(
        _mlp_body,
        out_shape=jax.ShapeDtypeStruct((N_GRAPHS, NUM_CLASSES), jnp.float32),
    )(sums, maxs, cnts, W1, b1[None, :], Wr1, br1[None, :], Wr2, br2[None, :],
      W2, b2[None, :], W3, b3[None, :])


def kernel(x_component, x_pin, x_net, x_subcircuit, e_cp, e_pc, e_sp, e_ps,
           e_pn, e_np, batch, node_type_emb, comp_type_emb, pin_type_emb,
           conv_Wl, conv_bl, conv_Wr, W1, b1, Wr1, br1, Wr2, br2, W2, b2,
           W3, b3):
    f32 = jnp.float32
    tbl = jnp.concatenate(
        [node_type_emb, comp_type_emb, pin_type_emb,
         jnp.zeros((32 - 4 - 9 - 13, H), f32)], axis=0)

    def cols(xa):
        return xa[:, 0:1], xa[:, 1:2], xa[:, 2:3]

    nt, _, pt = cols(x_component)
    x = {"component": _embed_tc(nt, jnp.zeros_like(nt), pt, tbl, NC_PAD)}
    for t, xa, n in (("pin", x_pin, N_PIN), ("net", x_net, N_NET),
                     ("subcircuit", x_subcircuit, N_SUB)):
        nt, ct, pt = cols(xa)
        x[t] = _embed_tc(nt, ct, pt, tbl, n)

    eid = []
    for (nm, _, _, ne), ei in zip(RELS, (e_cp, e_pc, e_sp, e_ps, e_pn, e_np)):
        p = _epad(ne) - ne
        s = jnp.concatenate([ei[0], jnp.zeros((p,), jnp.int32)])
        d = jnp.concatenate([ei[1], jnp.full((p,), -1, jnp.int32)])
        eid += [s, d]

    z128 = jnp.zeros((128, H), f32)
    o128 = jnp.ones((128, H), f32)

    cnts = _count_kernel(*(eid + [z128, o128]))
    for layer in range(NUM_LAYERS):
        args = [x["component"], x["pin"], x["net"], x["subcircuit"]] + eid + \
               [z128]
        accs = _sum_kernel(*args)
        new_x = {}
        for dt, ridx in DST_RELS.items():
            n = NC_PAD if dt == "component" else N_NODES[dt]
            wr = sum(conv_Wr[layer, r] for r in ridx)
            bl = sum(conv_bl[layer, r] for r in ridx)
            s = _self_tc(x[dt], wr, bl.reshape(1, H), n)
            new_x[dt] = _mix_tc(s, [accs[r] for r in ridx],
                                [cnts[r] for r in ridx],
                                [conv_Wl[layer, r] for r in ridx], n)
        x = new_x

    batch_pad = jnp.concatenate(
        [batch.astype(jnp.int32),
         jnp.full((NC_PAD - N_COMP,), N_GRAPHS, jnp.int32)])
    sums, maxs, pcnts = _pool_kernel(x["component"], batch_pad)
    return _mlp_tc(sums, maxs, pcnts, W1, b1, Wr1, br1, Wr2, br2, W2, b2,
                   W3, b3)
